# Initial kernel scaffold; baseline (speedup 1.0000x reference)
import jax
import jax.numpy as jnp
from jax.experimental import pallas as pl


def kernel(x, pos, edge_index, batch, params):
    raise NotImplementedError("write your pallas kernel here")


# speedup vs baseline: 3.5061x; 3.5061x over previous
import functools

import jax
import jax.numpy as jnp
from jax import lax
from jax.experimental import pallas as pl
from jax.experimental.pallas import tpu as pltpu
from jax.experimental.pallas import tpu_sc as plsc

N = 10000
E = 320000
HID = 64
G = 192
NWKR = 32
EPW = E // NWKR
WIN = 400
NWIN = EPW // WIN
CH = 80
NCH = WIN // CH
NACC = 10240
ZROWS = NACC // 16

f32 = jnp.float32
i32 = jnp.int32

_sc_mesh = plsc.VectorSubcoreMesh(core_axis_name="c", subcore_axis_name="s")
_sc_params = pltpu.CompilerParams(needs_layout_passes=False,
                                  use_tc_tiling_on_sc=False)


@functools.partial(
    pl.kernel,
    mesh=_sc_mesh,
    out_type=(
        jax.ShapeDtypeStruct((E, HID), f32),
        jax.ShapeDtypeStruct((E // CH, 1, CH), f32),
    ),
    scratch_types=[
        pltpu.VMEM((N,), f32),
        pltpu.VMEM((N,), f32),
        pltpu.VMEM((N,), f32),
        pltpu.VMEM((NCH, 1, CH), i32),
        pltpu.VMEM((NCH, 1, CH), i32),
        pltpu.VMEM((WIN, HID), f32),
        pltpu.VMEM((WIN, HID), f32),
        pltpu.VMEM((NCH, 1, CH), f32),
        pltpu.SemaphoreType.DMA,
    ],
    compiler_params=_sc_params,
)
def _sc_gather(u_hbm, v_hbm, row2_hbm, col2_hbm, cx_hbm, cy_hbm, cz_hbm,
               s_out, rad_out,
               cx_v, cy_v, cz_v, row_v, col_v, u_v, v_v, rad_v, sem):
    cid = lax.axis_index("c")
    sid = lax.axis_index("s")
    wid = sid * 2 + cid
    pltpu.sync_copy(cx_hbm, cx_v)
    pltpu.sync_copy(cy_hbm, cy_v)
    pltpu.sync_copy(cz_hbm, cz_v)

    def win_body(w, carry):
        eb = wid * EPW + w * WIN
        rb = wid * (EPW // CH) + w * NCH
        pltpu.sync_copy(row2_hbm.at[pl.ds(rb, NCH)], row_v)
        pltpu.sync_copy(col2_hbm.at[pl.ds(rb, NCH)], col_v)
        cps = []
        for c in range(NCH):
            cps.append(pltpu.async_copy(
                u_hbm.at[row_v.at[c, 0]], u_v.at[pl.ds(c * CH, CH), :], sem))
            cps.append(pltpu.async_copy(
                v_hbm.at[col_v.at[c, 0]], v_v.at[pl.ds(c * CH, CH), :], sem))
        for cp in cps:
            cp.wait()
        for j in range(WIN // 16):
            c, o = j // (CH // 16), (j % (CH // 16)) * 16
            rv = row_v[c, 0, pl.ds(o, 16)]
            cv = col_v[c, 0, pl.ds(o, 16)]
            dx = plsc.load_gather(cx_v, [rv]) - plsc.load_gather(cx_v, [cv])
            dy = plsc.load_gather(cy_v, [rv]) - plsc.load_gather(cy_v, [cv])
            dz = plsc.load_gather(cz_v, [rv]) - plsc.load_gather(cz_v, [cv])
            rad_v[c, 0, pl.ds(o, 16)] = dx * dx + dy * dy + dz * dz

        def add_row(e, acc):
            for q in range(HID // 16):
                u_v[e, pl.ds(q * 16, 16)] = (
                    u_v[e, pl.ds(q * 16, 16)] + v_v[e, pl.ds(q * 16, 16)])
            return acc
        lax.fori_loop(0, WIN, add_row, 0)
        pltpu.sync_copy(u_v, s_out.at[pl.ds(eb, WIN), :])
        pltpu.sync_copy(rad_v, rad_out.at[pl.ds(rb, NCH)])
        return carry

    lax.fori_loop(0, NWIN, win_body, 0)


@functools.partial(
    pl.kernel,
    mesh=_sc_mesh,
    out_type=(
        jax.ShapeDtypeStruct((2, NACC, HID), f32),
        jax.ShapeDtypeStruct((2, 1, NACC), f32),
        jax.ShapeDtypeStruct((2, 1, NACC), f32),
        jax.ShapeDtypeStruct((2, 1, NACC), f32),
        jax.ShapeDtypeStruct((2, 1, NACC), f32),
    ),
    scratch_types=[
        pltpu.VMEM((N,), f32),
        pltpu.VMEM((N,), f32),
        pltpu.VMEM((N,), f32),
        pltpu.VMEM((NCH, 1, CH), i32),
        pltpu.VMEM((NCH, 1, CH), i32),
        pltpu.VMEM((NCH, 1, CH), f32),
        pltpu.VMEM((WIN, HID), f32),
        pltpu.VMEM((NCH, 1, CH), f32),
        pltpu.VMEM((NCH, 1, CH), f32),
        pltpu.VMEM((NCH, 1, CH), f32),
        pltpu.VMEM((NCH, 1, CH), f32),
        pltpu.VMEM_SHARED((NACC, HID), f32),
        pltpu.VMEM_SHARED((NACC,), f32),
        pltpu.VMEM_SHARED((NACC,), f32),
        pltpu.VMEM_SHARED((NACC,), f32),
        pltpu.VMEM_SHARED((NACC,), f32),
        pltpu.SemaphoreType.DMA,
    ],
    compiler_params=_sc_params,
)
def _sc_scatter(m_hbm, w2_hbm, row2_hbm, col2_hbm, cx_hbm, cy_hbm, cz_hbm,
                z64_hbm, z1_hbm,
                agg_out, tx_out, ty_out, tz_out, tc_out,
                cx_v, cy_v, cz_v, row_v, col_v, w_v, m_v,
                txv, tyv, tzv, onev,
                agg_acc, tx_acc, ty_acc, tz_acc, tc_acc, sem):
    cid = lax.axis_index("c")
    sid = lax.axis_index("s")
    wid = sid * 2 + cid
    pltpu.sync_copy(cx_hbm, cx_v)
    pltpu.sync_copy(cy_hbm, cy_v)
    pltpu.sync_copy(cz_hbm, cz_v)
    pltpu.sync_copy(z64_hbm, agg_acc.at[pl.ds(sid * ZROWS, ZROWS), :])
    pltpu.sync_copy(z1_hbm, tx_acc.at[pl.ds(sid * ZROWS, ZROWS)])
    pltpu.sync_copy(z1_hbm, ty_acc.at[pl.ds(sid * ZROWS, ZROWS)])
    pltpu.sync_copy(z1_hbm, tz_acc.at[pl.ds(sid * ZROWS, ZROWS)])
    pltpu.sync_copy(z1_hbm, tc_acc.at[pl.ds(sid * ZROWS, ZROWS)])
    for c in range(NCH):
        for o in range(CH // 16):
            onev[c, 0, pl.ds(o * 16, 16)] = jnp.full((16,), 1.0, f32)
    plsc.subcore_barrier()

    def win_body(w, carry):
        eb = wid * EPW + w * WIN
        rb = wid * (EPW // CH) + w * NCH
        pltpu.sync_copy(row2_hbm.at[pl.ds(rb, NCH)], row_v)
        pltpu.sync_copy(col2_hbm.at[pl.ds(rb, NCH)], col_v)
        pltpu.sync_copy(w2_hbm.at[pl.ds(rb, NCH)], w_v)
        pltpu.async_copy(m_hbm.at[pl.ds(eb, WIN), :], m_v, sem).wait()
        for j in range(WIN // 16):
            c, o = j // (CH // 16), (j % (CH // 16)) * 16
            rv = row_v[c, 0, pl.ds(o, 16)]
            cv = col_v[c, 0, pl.ds(o, 16)]
            wv = w_v[c, 0, pl.ds(o, 16)]
            dx = plsc.load_gather(cx_v, [rv]) - plsc.load_gather(cx_v, [cv])
            dy = plsc.load_gather(cy_v, [rv]) - plsc.load_gather(cy_v, [cv])
            dz = plsc.load_gather(cz_v, [rv]) - plsc.load_gather(cz_v, [cv])
            txv[c, 0, pl.ds(o, 16)] = dx * wv
            tyv[c, 0, pl.ds(o, 16)] = dy * wv
            tzv[c, 0, pl.ds(o, 16)] = dz * wv
        for c in range(NCH):
            pltpu.sync_copy(m_v.at[pl.ds(c * CH, CH), :],
                            agg_acc.at[row_v.at[c, 0]], add=True)
            pltpu.sync_copy(txv.at[c, 0], tx_acc.at[row_v.at[c, 0]], add=True)
            pltpu.sync_copy(tyv.at[c, 0], ty_acc.at[row_v.at[c, 0]], add=True)
            pltpu.sync_copy(tzv.at[c, 0], tz_acc.at[row_v.at[c, 0]], add=True)
            pltpu.sync_copy(onev.at[c, 0], tc_acc.at[row_v.at[c, 0]], add=True)
        return carry

    lax.fori_loop(0, NWIN, win_body, 0)
    plsc.subcore_barrier()
    sl = pl.ds(sid * ZROWS, ZROWS)
    pltpu.sync_copy(agg_acc.at[sl, :], agg_out.at[cid, sl, :])
    pltpu.sync_copy(tx_acc.at[sl], tx_out.at[cid, 0, sl])
    pltpu.sync_copy(ty_acc.at[sl], ty_out.at[cid, 0, sl])
    pltpu.sync_copy(tz_acc.at[sl], tz_out.at[cid, 0, sl])
    pltpu.sync_copy(tc_acc.at[sl], tc_out.at[cid, 0, sl])


BN = 2000
BE = 4000
NPAD = 10240
BP = 2048


def _tc_embed_body(x_ref, wemb_ref, bemb_ref, w1a_ref, w1b_ref,
                   h_ref, u_ref, v_ref):
    h = jnp.dot(x_ref[...], wemb_ref[...], preferred_element_type=f32) + bemb_ref[...]
    h_ref[...] = h
    u_ref[...] = jnp.dot(h, w1a_ref[...], preferred_element_type=f32)
    v_ref[...] = jnp.dot(h, w1b_ref[...], preferred_element_type=f32)


_embed_call = pl.pallas_call(
    _tc_embed_body,
    grid=(N // BN,),
    in_specs=[
        pl.BlockSpec((BN, 128), lambda i: (i, 0)),
        pl.BlockSpec((128, HID), lambda i: (0, 0)),
        pl.BlockSpec((1, HID), lambda i: (0, 0)),
        pl.BlockSpec((HID, HID), lambda i: (0, 0)),
        pl.BlockSpec((HID, HID), lambda i: (0, 0)),
    ],
    out_specs=[
        pl.BlockSpec((BN, HID), lambda i: (i, 0)),
        pl.BlockSpec((BN, HID), lambda i: (i, 0)),
        pl.BlockSpec((BN, HID), lambda i: (i, 0)),
    ],
    out_shape=[
        jax.ShapeDtypeStruct((N, HID), f32),
        jax.ShapeDtypeStruct((N, HID), f32),
        jax.ShapeDtypeStruct((N, HID), f32),
    ],
)


def _tc_edge_body(s_ref, rad_ref, w1c_ref, b1_ref, w2_ref, b2_ref,
                  wc1_ref, bc1_ref, wc2_ref, m_ref, w_ref):
    p = jax.nn.silu(s_ref[...] + rad_ref[...] * w1c_ref[...] + b1_ref[...])
    m = jax.nn.silu(jnp.dot(p, w2_ref[...], preferred_element_type=f32) + b2_ref[...])
    m_ref[...] = m
    q = jax.nn.silu(jnp.dot(m, wc1_ref[...], preferred_element_type=f32) + bc1_ref[...])
    w_ref[...] = jnp.dot(q, wc2_ref[...], preferred_element_type=f32)


_edge_call = pl.pallas_call(
    _tc_edge_body,
    grid=(E // BE,),
    in_specs=[
        pl.BlockSpec((BE, HID), lambda i: (i, 0)),
        pl.BlockSpec((BE, 1), lambda i: (i, 0)),
        pl.BlockSpec((1, HID), lambda i: (0, 0)),
        pl.BlockSpec((1, HID), lambda i: (0, 0)),
        pl.BlockSpec((HID, HID), lambda i: (0, 0)),
        pl.BlockSpec((1, HID), lambda i: (0, 0)),
        pl.BlockSpec((HID, HID), lambda i: (0, 0)),
        pl.BlockSpec((1, HID), lambda i: (0, 0)),
        pl.BlockSpec((HID, 1), lambda i: (0, 0)),
    ],
    out_specs=[
        pl.BlockSpec((BE, HID), lambda i: (i, 0)),
        pl.BlockSpec((BE, 1), lambda i: (i, 0)),
    ],
    out_shape=[
        jax.ShapeDtypeStruct((E, HID), f32),
        jax.ShapeDtypeStruct((E, 1), f32),
    ],
)


def _tc_node_body(h_ref, a0_ref, a1_ref,
                  t0x_ref, t1x_ref, t0y_ref, t1y_ref, t0z_ref, t1z_ref,
                  t0c_ref, t1c_ref, cx_ref, cy_ref, cz_ref,
                  n1a_ref, n1b_ref, bn1_ref, n2_ref, bn2_ref,
                  wxa_ref, wxb_ref,
                  h_out, u_out, v_out, cx_out, cy_out, cz_out):
    agg = a0_ref[...] + a1_ref[...]
    cnt = jnp.clip(t0c_ref[...] + t1c_ref[...], 1.0, None)
    cx_out[...] = cx_ref[...] + (t0x_ref[...] + t1x_ref[...]) / cnt
    cy_out[...] = cy_ref[...] + (t0y_ref[...] + t1y_ref[...]) / cnt
    cz_out[...] = cz_ref[...] + (t0z_ref[...] + t1z_ref[...]) / cnt
    pre = (jnp.dot(h_ref[...], n1a_ref[...], preferred_element_type=f32)
           + jnp.dot(agg, n1b_ref[...], preferred_element_type=f32) + bn1_ref[...])
    hn = h_ref[...] + jnp.dot(jax.nn.silu(pre), n2_ref[...],
                              preferred_element_type=f32) + bn2_ref[...]
    h_out[...] = hn
    u_out[...] = jnp.dot(hn, wxa_ref[...], preferred_element_type=f32)
    v_out[...] = jnp.dot(hn, wxb_ref[...], preferred_element_type=f32)


def _tc_node_last_body(h_ref, a0_ref, a1_ref,
                       n1a_ref, n1b_ref, bn1_ref, n2_ref, bn2_ref,
                       wout_ref, bout_ref, henc_out):
    agg = a0_ref[...] + a1_ref[...]
    pre = (jnp.dot(h_ref[...], n1a_ref[...], preferred_element_type=f32)
           + jnp.dot(agg, n1b_ref[...], preferred_element_type=f32) + bn1_ref[...])
    hn = h_ref[...] + jnp.dot(jax.nn.silu(pre), n2_ref[...],
                              preferred_element_type=f32) + bn2_ref[...]
    henc_out[...] = jnp.dot(hn, wout_ref[...], preferred_element_type=f32) + bout_ref[...]


_mat = lambda: pl.BlockSpec((HID, HID), lambda i: (0, 0))
_vec = lambda: pl.BlockSpec((1, HID), lambda i: (0, 0))
_nblk = lambda w=HID: pl.BlockSpec((BN, w), lambda i: (i, 0))

_node_call = pl.pallas_call(
    _tc_node_body,
    grid=(N // BN,),
    in_specs=[_nblk(), _nblk(), _nblk(),
              _nblk(1), _nblk(1), _nblk(1), _nblk(1), _nblk(1), _nblk(1),
              _nblk(1), _nblk(1), _nblk(1), _nblk(1), _nblk(1),
              _mat(), _mat(), _vec(), _mat(), _vec(), _mat(), _mat()],
    out_specs=[_nblk(), _nblk(), _nblk(), _nblk(1), _nblk(1), _nblk(1)],
    out_shape=[
        jax.ShapeDtypeStruct((N, HID), f32),
        jax.ShapeDtypeStruct((N, HID), f32),
        jax.ShapeDtypeStruct((N, HID), f32),
        jax.ShapeDtypeStruct((N, 1), f32),
        jax.ShapeDtypeStruct((N, 1), f32),
        jax.ShapeDtypeStruct((N, 1), f32),
    ],
)

_node_last_call = pl.pallas_call(
    _tc_node_last_body,
    grid=(N // BN,),
    in_specs=[_nblk(), _nblk(), _nblk(),
              _mat(), _mat(), _vec(), _mat(), _vec(), _mat(), _vec()],
    out_specs=[_nblk()],
    out_shape=[jax.ShapeDtypeStruct((N, HID), f32)],
)


def _tc_pool_body(he_ref, b_ref, gs_ref, gc_ref):
    @pl.when(pl.program_id(0) == 0)
    def _():
        gs_ref[...] = jnp.zeros_like(gs_ref)
        gc_ref[...] = jnp.zeros_like(gc_ref)
    oh = (b_ref[...] == lax.broadcasted_iota(i32, (G, BP), 0)).astype(f32)
    gs_ref[...] += jnp.dot(oh, he_ref[...], preferred_element_type=f32)
    gc_ref[...] += jnp.dot(oh, jnp.ones((BP, 1), f32), preferred_element_type=f32)


_pool_call = pl.pallas_call(
    _tc_pool_body,
    grid=(NPAD // BP,),
    in_specs=[
        pl.BlockSpec((BP, HID), lambda i: (i, 0)),
        pl.BlockSpec((1, BP), lambda i: (0, i)),
    ],
    out_specs=[
        pl.BlockSpec((G, HID), lambda i: (0, 0)),
        pl.BlockSpec((G, 1), lambda i: (0, 0)),
    ],
    out_shape=[
        jax.ShapeDtypeStruct((G, HID), f32),
        jax.ShapeDtypeStruct((G, 1), f32),
    ],
)


def _tc_head_body(gs_ref, gc_ref, wm_ref, bm_ref, wl_ref, bl_ref,
                  mean_ref, lv_ref):
    ge = gs_ref[...] / jnp.clip(gc_ref[...], 1.0, None)
    mean_ref[...] = jnp.dot(ge, wm_ref[...], preferred_element_type=f32) + bm_ref[...]
    lv_ref[...] = jnp.dot(ge, wl_ref[...], preferred_element_type=f32) + bl_ref[...]


_head_call = pl.pallas_call(
    _tc_head_body,
    out_shape=[
        jax.ShapeDtypeStruct((G, HID), f32),
        jax.ShapeDtypeStruct((G, HID), f32),
    ],
)


def kernel(x, pos, edge_index, batch, params):
    row2 = edge_index[0].reshape(E // CH, 1, CH)
    col2 = edge_index[1].reshape(E // CH, 1, CH)
    cx, cy, cz = pos[:, 0:1], pos[:, 1:2], pos[:, 2:3]
    z64 = jnp.zeros((ZROWS, HID), f32)
    z1 = jnp.zeros((ZROWS,), f32)
    batch_row = jnp.pad(batch, (0, NPAD - N), constant_values=255).reshape(1, NPAD)

    lps = params["layers"]
    w1a = [lp["edge1"]["W"][:HID] for lp in lps]
    w1b = [lp["edge1"]["W"][HID:2 * HID] for lp in lps]
    w1c = [lp["edge1"]["W"][2 * HID].reshape(1, HID) for lp in lps]
    b1 = [lp["edge1"]["b"].reshape(1, HID) for lp in lps]

    h, u, v = _embed_call(
        x, params["emb_in"]["W"], params["emb_in"]["b"].reshape(1, HID),
        w1a[0], w1b[0])

    for l, lp in enumerate(lps):
        cx1, cy1, cz1 = cx.reshape(N), cy.reshape(N), cz.reshape(N)
        s, rad = _sc_gather(u, v, row2, col2, cx1, cy1, cz1)
        m, w = _edge_call(
            s, rad.reshape(E, 1), w1c[l], b1[l],
            lp["edge2"]["W"], lp["edge2"]["b"].reshape(1, HID),
            lp["coord1"]["W"], lp["coord1"]["b"].reshape(1, HID),
            lp["coord2"]["W"])
        aggp, txp, typ, tzp, tcp = _sc_scatter(
            m, w.reshape(E // CH, 1, CH), row2, col2, cx1, cy1, cz1, z64, z1)
        if l < 3:
            h, u, v, cx, cy, cz = _node_call(
                h, aggp[0, :N], aggp[1, :N],
                txp[0, 0, :N].reshape(N, 1), txp[1, 0, :N].reshape(N, 1),
                typ[0, 0, :N].reshape(N, 1), typ[1, 0, :N].reshape(N, 1),
                tzp[0, 0, :N].reshape(N, 1), tzp[1, 0, :N].reshape(N, 1),
                tcp[0, 0, :N].reshape(N, 1), tcp[1, 0, :N].reshape(N, 1),
                cx, cy, cz,
                lp["node1"]["W"][:HID], lp["node1"]["W"][HID:],
                lp["node1"]["b"].reshape(1, HID),
                lp["node2"]["W"], lp["node2"]["b"].reshape(1, HID),
                w1a[l + 1], w1b[l + 1])
        else:
            (h_enc,) = _node_last_call(
                h, aggp[0, :N], aggp[1, :N],
                lp["node1"]["W"][:HID], lp["node1"]["W"][HID:],
                lp["node1"]["b"].reshape(1, HID),
                lp["node2"]["W"], lp["node2"]["b"].reshape(1, HID),
                params["emb_out"]["W"], params["emb_out"]["b"].reshape(1, HID))

    he_pad = jnp.pad(h_enc, ((0, NPAD - N), (0, 0)))
    gs, gc = _pool_call(he_pad, batch_row)
    mean, log_var = _head_call(
        gs, gc, params["fc_mean"]["W"], params["fc_mean"]["b"].reshape(1, HID),
        params["fc_log_var"]["W"], params["fc_log_var"]["b"].reshape(1, HID))
    return (mean, log_var)
